# baseline (speedup 1.0000x reference)
import functools

import jax
import jax.numpy as jnp
from jax import lax
from jax.experimental import pallas as pl
from jax.experimental.pallas import tpu as pltpu
from jax.experimental.pallas import tpu_sc as plsc

_N = 10000
_E = 320000
_NC = 2
_NS = 16
_NW = _NC * _NS
_CH = 64
_SB = 32
_CPT0 = 224
_CPT1 = 96
_NSB0 = _CPT0 // _SB
_NSB1 = _CPT1 // _SB
_SRCW = _CPT0 * _CH
_E0 = _CPT0 * _CH * _NS
_E1PT = (_E - _E0) // _NS
_NPAD = 10240
_RPT = _NPAD // _NS
_BR = 512


def _seg_sum_call(F):
    mesh = plsc.VectorSubcoreMesh(core_axis_name="c", subcore_axis_name="s")

    def body(y_hbm, src_hbm, dst_hbm, out_hbm, src_v, dstb0, dstb1, rows0, rows1,
             acc, sem0, sem1, semd0, semd1):
        c = lax.axis_index("c")
        s = lax.axis_index("s")
        wid = c * _NS + s
        cpt = jnp.where(c == 0, _CPT0, _CPT1)
        nsb = jnp.where(c == 0, _NSB0, _NSB1)
        npair = jnp.where(c == 0, (_NSB0 - 1) // 2, (_NSB1 - 1) // 2)

        def gat(j, buf, sem):
            off = pl.multiple_of(j * _CH, _CH)
            return pltpu.make_async_copy(
                y_hbm.at[src_v.at[pl.ds(off, _CH)]], buf, sem)

        def dld(sb, buf, sem):
            return pltpu.make_async_copy(
                dst_hbm.at[wid, pl.ds(sb * _SB, _SB)], buf, sem)

        @pl.loop(0, _SRCW + 2 * _CH, step=16)
        def _(i):
            src_v[pl.ds(i, 16)] = jnp.zeros((16,), jnp.int32)

        @pl.loop(0, _CH)
        def _(i):
            @pl.loop(0, F, step=16)
            def _(l):
                rows0[i, pl.ds(l, 16)] = jnp.zeros((16,), jnp.float32)

        @pl.loop(0, _RPT // _CH)
        def _(r):
            pltpu.sync_copy(rows0, acc.at[pl.ds(s * _RPT + r * _CH, _CH)])

        pltpu.sync_copy(src_hbm.at[wid], src_v.at[pl.ds(0, _SRCW)])
        d0 = dld(0, dstb0, semd0)
        d0.start()
        d0.wait()
        dld(1, dstb1, semd1).start()
        plsc.subcore_barrier()

        gat(0, rows0, sem0).start()
        gat(1, rows1, sem1).start()

        def pairs(base, dstb):
            @pl.loop(0, _SB // 2)
            def _(k):
                j = base + 2 * k
                gat(j, rows0, sem0).wait()
                pltpu.sync_copy(rows0, acc.at[dstb.at[2 * k]], add=True)
                gat(j + 2, rows0, sem0).start()
                gat(j + 1, rows1, sem1).wait()
                pltpu.sync_copy(rows1, acc.at[dstb.at[2 * k + 1]], add=True)
                gat(j + 3, rows1, sem1).start()

        pairs(0, dstb0)
        dld(2, dstb0, semd0).start()

        @pl.loop(0, (_NSB0 - 1) // 2)
        def _(it):
            @pl.when(it < npair)
            def _():
                sba = 1 + 2 * it
                dld(sba, dstb1, semd1).wait()
                pairs(sba * _SB, dstb1)
                dld(jnp.minimum(sba + 2, nsb - 1), dstb1, semd1).start()
                sbb = sba + 1
                dld(sbb, dstb0, semd0).wait()
                pairs(sbb * _SB, dstb0)
                dld(jnp.minimum(sbb + 2, nsb - 1), dstb0, semd0).start()

        dld(0, dstb1, semd1).wait()
        dld(0, dstb0, semd0).wait()
        gat(cpt, rows0, sem0).wait()
        gat(cpt + 1, rows1, sem1).wait()

        plsc.subcore_barrier()
        pltpu.sync_copy(acc.at[pl.ds(s * _RPT, _RPT)],
                        out_hbm.at[c, pl.ds(s * _RPT, _RPT)])

    return pl.kernel(
        body,
        mesh=mesh,
        out_type=jax.ShapeDtypeStruct((_NC, _NPAD, F), jnp.float32),
        scratch_types=[
            pltpu.VMEM((_SRCW + 2 * _CH,), jnp.int32),
            pltpu.VMEM((_SB, _CH), jnp.int32),
            pltpu.VMEM((_SB, _CH), jnp.int32),
            pltpu.VMEM((_CH, F), jnp.float32),
            pltpu.VMEM((_CH, F), jnp.float32),
            pltpu.VMEM_SHARED((_NPAD, F), jnp.float32),
            pltpu.SemaphoreType.DMA,
            pltpu.SemaphoreType.DMA,
            pltpu.SemaphoreType.DMA,
            pltpu.SemaphoreType.DMA,
        ],
    )


def _count_call():
    mesh = plsc.VectorSubcoreMesh(core_axis_name="c", subcore_axis_name="s")

    def body(dst_hbm, out_hbm, dst_v, cnt_v):
        c = lax.axis_index("c")
        s = lax.axis_index("s")
        wid = c * _NS + s

        @pl.loop(0, _NPAD, step=16)
        def _(i):
            cnt_v[pl.ds(i, 16)] = jnp.zeros((16,), jnp.float32)

        pltpu.sync_copy(dst_hbm.at[wid], dst_v)

        @pl.loop(0, _CPT0)
        def _(j):
            @pl.loop(0, _CH, step=16)
            def _(l):
                idx = dst_v[j, pl.ds(l, 16)]
                plsc.addupdate_scatter(cnt_v, [idx], jnp.ones((16,), jnp.float32))

        pltpu.sync_copy(cnt_v, out_hbm.at[wid])

    return pl.kernel(
        body,
        mesh=mesh,
        out_type=jax.ShapeDtypeStruct((_NW, _NPAD), jnp.float32),
        scratch_types=[
            pltpu.VMEM((_CPT0, _CH), jnp.int32),
            pltpu.VMEM((_NPAD,), jnp.float32),
        ],
        compiler_params=pltpu.CompilerParams(needs_layout_passes=False),
    )


_DN = (((1,), (0,)), ((), ()))


def _prep_body(x_ref, w_ref, b_ref, y_ref, z_ref, *, Fo):
    r = lax.dot_general(x_ref[...], w_ref[...], _DN,
                        preferred_element_type=jnp.float32)
    y_ref[...] = r[:, :Fo]
    z_ref[...] = r[:, Fo:] + b_ref[...]


def _prep_call(x, wcat, b):
    Fi, F2 = wcat.shape
    Fo = F2 // 2
    return pl.pallas_call(
        functools.partial(_prep_body, Fo=Fo),
        grid=(-(-_N // _BR),),
        in_specs=[
            pl.BlockSpec((_BR, Fi), lambda i: (i, 0)),
            pl.BlockSpec((Fi, F2), lambda i: (0, 0)),
            pl.BlockSpec((1, Fo), lambda i: (0, 0)),
        ],
        out_specs=[
            pl.BlockSpec((_BR, Fo), lambda i: (i, 0)),
            pl.BlockSpec((_BR, Fo), lambda i: (i, 0)),
        ],
        out_shape=[jax.ShapeDtypeStruct((_N, Fo), jnp.float32)] * 2,
    )(x, wcat, b)


def _inv_cnt(c_ref):
    cnt = lax.dot_general(c_ref[...], jnp.ones((_NW, 1), jnp.float32),
                          (((0,), (0,)), ((), ())),
                          preferred_element_type=jnp.float32)
    return 1.0 / jnp.maximum(cnt, 1.0)


def _combine(p_ref, c_ref, z_ref):
    return (p_ref[0] + p_ref[1]) * _inv_cnt(c_ref) + z_ref[...]


def _mid_body(p_ref, c_ref, z_ref, w_ref, b_ref, y_ref, z2_ref, *, Fo):
    h = jnp.maximum(_combine(p_ref, c_ref, z_ref), 0.0)
    r = lax.dot_general(h, w_ref[...], _DN, preferred_element_type=jnp.float32)
    y_ref[...] = r[:, :Fo]
    z2_ref[...] = r[:, Fo:] + b_ref[...]


def _mid_call(p, pcnt, z, wcat, b):
    Fi, F2 = wcat.shape
    Fo = F2 // 2
    return pl.pallas_call(
        functools.partial(_mid_body, Fo=Fo),
        grid=(-(-_N // _BR),),
        in_specs=[
            pl.BlockSpec((_NC, _BR, Fi), lambda i: (0, i, 0)),
            pl.BlockSpec((_NW, _BR), lambda i: (0, i)),
            pl.BlockSpec((_BR, Fi), lambda i: (i, 0)),
            pl.BlockSpec((Fi, F2), lambda i: (0, 0)),
            pl.BlockSpec((1, Fo), lambda i: (0, 0)),
        ],
        out_specs=[
            pl.BlockSpec((_BR, Fo), lambda i: (i, 0)),
            pl.BlockSpec((_BR, Fo), lambda i: (i, 0)),
        ],
        out_shape=[jax.ShapeDtypeStruct((_N, Fo), jnp.float32)] * 2,
    )(p, pcnt, z, wcat, b)


def _mid2_body(p_ref, c_ref, z_ref, w_ref, b_ref, h_ref, z3_ref):
    h = jnp.maximum(_combine(p_ref, c_ref, z_ref), 0.0)
    h_ref[...] = h
    z3_ref[...] = lax.dot_general(h, w_ref[...], _DN,
                                  preferred_element_type=jnp.float32) + b_ref[...]


def _mid2_call(p, pcnt, z, wr, b):
    Fi, Fo = wr.shape
    return pl.pallas_call(
        _mid2_body,
        grid=(-(-_N // _BR),),
        in_specs=[
            pl.BlockSpec((_NC, _BR, Fi), lambda i: (0, i, 0)),
            pl.BlockSpec((_NW, _BR), lambda i: (0, i)),
            pl.BlockSpec((_BR, Fi), lambda i: (i, 0)),
            pl.BlockSpec((Fi, Fo), lambda i: (0, 0)),
            pl.BlockSpec((1, Fo), lambda i: (0, 0)),
        ],
        out_specs=[
            pl.BlockSpec((_BR, Fi), lambda i: (i, 0)),
            pl.BlockSpec((_BR, Fo), lambda i: (i, 0)),
        ],
        out_shape=[jax.ShapeDtypeStruct((_N, Fi), jnp.float32),
                   jax.ShapeDtypeStruct((_N, Fo), jnp.float32)],
    )(p, pcnt, z, wr, b)


def _final_body(p_ref, c_ref, z3_ref, w_ref, o_ref):
    mean = (p_ref[0] + p_ref[1]) * _inv_cnt(c_ref)
    o_ref[...] = lax.dot_general(mean, w_ref[...], _DN,
                                 preferred_element_type=jnp.float32) + z3_ref[...]


def _final_call(p, pcnt, z3, wl):
    Fi, Fo = wl.shape
    return pl.pallas_call(
        _final_body,
        grid=(-(-_N // _BR),),
        in_specs=[
            pl.BlockSpec((_NC, _BR, Fi), lambda i: (0, i, 0)),
            pl.BlockSpec((_NW, _BR), lambda i: (0, i)),
            pl.BlockSpec((_BR, Fo), lambda i: (i, 0)),
            pl.BlockSpec((Fi, Fo), lambda i: (0, 0)),
        ],
        out_specs=pl.BlockSpec((_BR, Fo), lambda i: (i, 0)),
        out_shape=jax.ShapeDtypeStruct((_N, Fo), jnp.float32),
    )(p, pcnt, z3, wl)


def kernel(x, edge_index, Wl1, bl1, Wr1, Wl2, bl2, Wr2, Wl3, bl3, Wr3):
    src = edge_index[0].astype(jnp.int32)
    dst = edge_index[1].astype(jnp.int32)
    s0 = src[:_E0].reshape(_NS, _SRCW)
    s1 = jnp.concatenate(
        [src[_E0:].reshape(_NS, _E1PT),
         jnp.zeros((_NS, _SRCW - _E1PT), jnp.int32)], axis=1)
    srcp = jnp.concatenate([s0, s1], axis=0)
    d0 = dst[:_E0].reshape(_NS, _CPT0, _CH)
    d1 = jnp.concatenate(
        [dst[_E0:].reshape(_NS, _E1PT),
         jnp.full((_NS, _CPT0 * _CH - _E1PT), _N, jnp.int32)],
        axis=1).reshape(_NS, _CPT0, _CH)
    dstp = jnp.concatenate([d0, d1], axis=0)

    w1 = jnp.concatenate([Wl1, Wr1], axis=1)
    w2 = jnp.concatenate([Wl2, Wr2], axis=1)
    b1, b2, b3 = (b.reshape(1, -1) for b in (bl1, bl2, bl3))

    pcnt = _count_call()(dstp)
    y1, z1 = _prep_call(x, w1, b1)
    p1 = _seg_sum_call(128)(y1, srcp, dstp)
    y2, z2 = _mid_call(p1, pcnt, z1, w2, b2)
    p2 = _seg_sum_call(128)(y2, srcp, dstp)
    h2, z3 = _mid2_call(p2, pcnt, z2, Wr3, b3)
    p3 = _seg_sum_call(128)(h2, srcp, dstp)
    return _final_call(p3, pcnt, z3, Wl3)


# speedup vs baseline: 2.7403x; 2.7403x over previous
import functools

import jax
import jax.numpy as jnp
from jax import lax
from jax.experimental import pallas as pl
from jax.experimental.pallas import tpu as pltpu
from jax.experimental.pallas import tpu_sc as plsc

_N = 10000
_E = 320000
_NC = 2
_NS = 16
_NW = _NC * _NS
_CH = 64
_CPT0 = 174
_CPT1 = 140
_SRCW = _CPT0 * _CH
_E0 = _CPT0 * _CH * _NS
_E1PT = (_E - _E0) // _NS
_NPAD = 10112
_RPT = _NPAD // _NS
_BR = 512


def _seg_sum_call(F):
    mesh = plsc.VectorSubcoreMesh(core_axis_name="c", subcore_axis_name="s")

    def body(y_hbm, src_hbm, dst_hbm, out_hbm, src_v, dst_v, rows0, rows1, acc,
             sem0, sem1, sem2, sem3):
        c = lax.axis_index("c")
        s = lax.axis_index("s")
        wid = c * _NS + s
        cpt = jnp.where(c == 0, _CPT0, _CPT1)
        npair = jnp.where(c == 0, (_CPT0 - 2) // 2, (_CPT1 - 2) // 2)

        def gat(j, buf, sem):
            off = pl.multiple_of(j * _CH, _CH)
            return pltpu.make_async_copy(
                y_hbm.at[src_v.at[pl.ds(off, _CH)]], buf, sem)

        @pl.loop(0, _CH)
        def _(i):
            @pl.loop(0, F, step=16)
            def _(l):
                rows0[i, pl.ds(l, 16)] = jnp.zeros((16,), jnp.float32)

        @pl.loop(0, _RPT // _CH)
        def _(r):
            pltpu.sync_copy(rows0, acc.at[pl.ds(s * _RPT + r * _CH, _CH)])

        rem = _RPT - (_RPT // _CH) * _CH
        pltpu.sync_copy(rows0.at[pl.ds(0, rem)],
                        acc.at[pl.ds(s * _RPT + _RPT - rem, rem)])

        pltpu.sync_copy(src_hbm.at[wid], src_v)
        pltpu.sync_copy(dst_hbm.at[wid], dst_v)
        plsc.subcore_barrier()

        gat(0, rows0, sem0).start()
        gat(1, rows1, sem1).start()

        @pl.loop(0, (_CPT0 - 2) // 2)
        def _(it):
            @pl.when(it < npair)
            def _():
                j = it * 2
                gat(j, rows0, sem0).wait()
                pltpu.sync_copy(rows0, acc.at[dst_v.at[j]], add=True)
                gat(j + 2, rows0, sem0).start()
                gat(j + 1, rows1, sem1).wait()
                pltpu.sync_copy(rows1, acc.at[dst_v.at[j + 1]], add=True)
                gat(j + 3, rows1, sem1).start()

        gat(cpt - 2, rows0, sem0).wait()
        pltpu.sync_copy(rows0, acc.at[dst_v.at[cpt - 2]], add=True)
        gat(cpt - 1, rows1, sem1).wait()
        pltpu.sync_copy(rows1, acc.at[dst_v.at[cpt - 1]], add=True)

        plsc.subcore_barrier()
        pltpu.sync_copy(acc.at[pl.ds(s * _RPT, _RPT)],
                        out_hbm.at[c, pl.ds(s * _RPT, _RPT)])

    return pl.kernel(
        body,
        mesh=mesh,
        out_type=jax.ShapeDtypeStruct((_NC, _NPAD, F), jnp.float32),
        scratch_types=[
            pltpu.VMEM((_SRCW,), jnp.int32),
            pltpu.VMEM((_CPT0, _CH), jnp.int32),
            pltpu.VMEM((_CH, F), jnp.float32),
            pltpu.VMEM((_CH, F), jnp.float32),
            pltpu.VMEM_SHARED((_NPAD, F), jnp.float32),
            pltpu.SemaphoreType.DMA,
            pltpu.SemaphoreType.DMA,
            pltpu.SemaphoreType.DMA,
            pltpu.SemaphoreType.DMA,
        ],
    )


def _count_call():
    mesh = plsc.VectorSubcoreMesh(core_axis_name="c", subcore_axis_name="s")

    def body(dst_hbm, out_hbm, dst_v, cnt_v):
        c = lax.axis_index("c")
        s = lax.axis_index("s")
        wid = c * _NS + s

        @pl.loop(0, _NPAD, step=16)
        def _(i):
            cnt_v[pl.ds(i, 16)] = jnp.zeros((16,), jnp.float32)

        pltpu.sync_copy(dst_hbm.at[wid], dst_v)

        @pl.loop(0, _CPT0)
        def _(j):
            @pl.loop(0, _CH, step=16)
            def _(l):
                idx = dst_v[j, pl.ds(l, 16)]
                plsc.addupdate_scatter(cnt_v, [idx], jnp.ones((16,), jnp.float32))

        pltpu.sync_copy(cnt_v, out_hbm.at[wid])

    return pl.kernel(
        body,
        mesh=mesh,
        out_type=jax.ShapeDtypeStruct((_NW, _NPAD), jnp.float32),
        scratch_types=[
            pltpu.VMEM((_CPT0, _CH), jnp.int32),
            pltpu.VMEM((_NPAD,), jnp.float32),
        ],
        compiler_params=pltpu.CompilerParams(needs_layout_passes=False),
    )


_DN = (((1,), (0,)), ((), ()))


def _prep_body(x_ref, w_ref, b_ref, y_ref, z_ref, *, Fo):
    r = lax.dot_general(x_ref[...], w_ref[...], _DN,
                        preferred_element_type=jnp.float32)
    y_ref[...] = r[:, :Fo]
    z_ref[...] = r[:, Fo:] + b_ref[...]


def _prep_call(x, wcat, b):
    Fi, F2 = wcat.shape
    Fo = F2 // 2
    return pl.pallas_call(
        functools.partial(_prep_body, Fo=Fo),
        grid=(-(-_N // _BR),),
        in_specs=[
            pl.BlockSpec((_BR, Fi), lambda i: (i, 0)),
            pl.BlockSpec((Fi, F2), lambda i: (0, 0)),
            pl.BlockSpec((1, Fo), lambda i: (0, 0)),
        ],
        out_specs=[
            pl.BlockSpec((_BR, Fo), lambda i: (i, 0)),
            pl.BlockSpec((_BR, Fo), lambda i: (i, 0)),
        ],
        out_shape=[jax.ShapeDtypeStruct((_N, Fo), jnp.float32)] * 2,
    )(x, wcat, b)


def _inv_cnt(c_ref):
    cnt = lax.dot_general(c_ref[...], jnp.ones((_NW, 1), jnp.float32),
                          (((0,), (0,)), ((), ())),
                          preferred_element_type=jnp.float32)
    return 1.0 / jnp.maximum(cnt, 1.0)


def _combine(p_ref, c_ref, z_ref):
    return (p_ref[0] + p_ref[1]) * _inv_cnt(c_ref) + z_ref[...]


def _mid_body(p_ref, c_ref, z_ref, w_ref, b_ref, y_ref, z2_ref, *, Fo):
    h = jnp.maximum(_combine(p_ref, c_ref, z_ref), 0.0)
    r = lax.dot_general(h, w_ref[...], _DN, preferred_element_type=jnp.float32)
    y_ref[...] = r[:, :Fo]
    z2_ref[...] = r[:, Fo:] + b_ref[...]


def _mid_call(p, pcnt, z, wcat, b):
    Fi, F2 = wcat.shape
    Fo = F2 // 2
    return pl.pallas_call(
        functools.partial(_mid_body, Fo=Fo),
        grid=(-(-_N // _BR),),
        in_specs=[
            pl.BlockSpec((_NC, _BR, Fi), lambda i: (0, i, 0)),
            pl.BlockSpec((_NW, _BR), lambda i: (0, i)),
            pl.BlockSpec((_BR, Fi), lambda i: (i, 0)),
            pl.BlockSpec((Fi, F2), lambda i: (0, 0)),
            pl.BlockSpec((1, Fo), lambda i: (0, 0)),
        ],
        out_specs=[
            pl.BlockSpec((_BR, Fo), lambda i: (i, 0)),
            pl.BlockSpec((_BR, Fo), lambda i: (i, 0)),
        ],
        out_shape=[jax.ShapeDtypeStruct((_N, Fo), jnp.float32)] * 2,
    )(p, pcnt, z, wcat, b)


def _mid2_body(p_ref, c_ref, z_ref, w_ref, b_ref, h_ref, z3_ref):
    h = jnp.maximum(_combine(p_ref, c_ref, z_ref), 0.0)
    h_ref[...] = h
    z3_ref[...] = lax.dot_general(h, w_ref[...], _DN,
                                  preferred_element_type=jnp.float32) + b_ref[...]


def _mid2_call(p, pcnt, z, wr, b):
    Fi, Fo = wr.shape
    return pl.pallas_call(
        _mid2_body,
        grid=(-(-_N // _BR),),
        in_specs=[
            pl.BlockSpec((_NC, _BR, Fi), lambda i: (0, i, 0)),
            pl.BlockSpec((_NW, _BR), lambda i: (0, i)),
            pl.BlockSpec((_BR, Fi), lambda i: (i, 0)),
            pl.BlockSpec((Fi, Fo), lambda i: (0, 0)),
            pl.BlockSpec((1, Fo), lambda i: (0, 0)),
        ],
        out_specs=[
            pl.BlockSpec((_BR, Fi), lambda i: (i, 0)),
            pl.BlockSpec((_BR, Fo), lambda i: (i, 0)),
        ],
        out_shape=[jax.ShapeDtypeStruct((_N, Fi), jnp.float32),
                   jax.ShapeDtypeStruct((_N, Fo), jnp.float32)],
    )(p, pcnt, z, wr, b)


def _final_body(p_ref, c_ref, z3_ref, w_ref, o_ref):
    mean = (p_ref[0] + p_ref[1]) * _inv_cnt(c_ref)
    o_ref[...] = lax.dot_general(mean, w_ref[...], _DN,
                                 preferred_element_type=jnp.float32) + z3_ref[...]


def _final_call(p, pcnt, z3, wl):
    Fi, Fo = wl.shape
    return pl.pallas_call(
        _final_body,
        grid=(-(-_N // _BR),),
        in_specs=[
            pl.BlockSpec((_NC, _BR, Fi), lambda i: (0, i, 0)),
            pl.BlockSpec((_NW, _BR), lambda i: (0, i)),
            pl.BlockSpec((_BR, Fo), lambda i: (i, 0)),
            pl.BlockSpec((Fi, Fo), lambda i: (0, 0)),
        ],
        out_specs=pl.BlockSpec((_BR, Fo), lambda i: (i, 0)),
        out_shape=jax.ShapeDtypeStruct((_N, Fo), jnp.float32),
    )(p, pcnt, z3, wl)


def kernel(x, edge_index, Wl1, bl1, Wr1, Wl2, bl2, Wr2, Wl3, bl3, Wr3):
    src = edge_index[0].astype(jnp.int32)
    dst = edge_index[1].astype(jnp.int32)
    s0 = src[:_E0].reshape(_NS, _SRCW)
    s1 = jnp.concatenate(
        [src[_E0:].reshape(_NS, _E1PT),
         jnp.zeros((_NS, _SRCW - _E1PT), jnp.int32)], axis=1)
    srcp = jnp.concatenate([s0, s1], axis=0)
    d0 = dst[:_E0].reshape(_NS, _CPT0, _CH)
    d1 = jnp.concatenate(
        [dst[_E0:].reshape(_NS, _E1PT),
         jnp.full((_NS, _SRCW - _E1PT), _N, jnp.int32)],
        axis=1).reshape(_NS, _CPT0, _CH)
    dstp = jnp.concatenate([d0, d1], axis=0)

    w1 = jnp.concatenate([Wl1, Wr1], axis=1)
    w2 = jnp.concatenate([Wl2, Wr2], axis=1)
    b1, b2, b3 = (b.reshape(1, -1) for b in (bl1, bl2, bl3))

    pcnt = _count_call()(dstp)
    y1, z1 = _prep_call(x, w1, b1)
    p1 = _seg_sum_call(128)(y1, srcp, dstp)
    y2, z2 = _mid_call(p1, pcnt, z1, w2, b2)
    p2 = _seg_sum_call(128)(y2, srcp, dstp)
    h2, z3 = _mid2_call(p2, pcnt, z2, Wr3, b3)
    p3 = _seg_sum_call(128)(h2, srcp, dstp)
    return _final_call(p3, pcnt, z3, Wl3)


# speedup vs baseline: 2.8772x; 1.0499x over previous
import functools

import jax
import jax.numpy as jnp
from jax import lax
from jax.experimental import pallas as pl
from jax.experimental.pallas import tpu as pltpu
from jax.experimental.pallas import tpu_sc as plsc

_N = 10000
_E = 320000
_NC = 2
_NS = 16
_NW = _NC * _NS
_CH = 64
_CPT0 = 174
_CPT1 = 140
_SRCW = _CPT0 * _CH
_E0 = _CPT0 * _CH * _NS
_E1PT = (_E - _E0) // _NS
_NPAD = 10112
_RPT = _NPAD // _NS
_BR = 2048


def _seg_sum_call(F):
    mesh = plsc.VectorSubcoreMesh(core_axis_name="c", subcore_axis_name="s")

    def body(y_hbm, src_hbm, dst_hbm, out_hbm, src_v, dst_v, rows0, rows1, acc,
             sem0, sem1, sem2, sem3):
        c = lax.axis_index("c")
        s = lax.axis_index("s")
        wid = c * _NS + s
        cpt = jnp.where(c == 0, _CPT0, _CPT1)
        npair = jnp.where(c == 0, (_CPT0 - 2) // 2, (_CPT1 - 2) // 2)

        def gat(j, buf, sem):
            off = pl.multiple_of(j * _CH, _CH)
            return pltpu.make_async_copy(
                y_hbm.at[src_v.at[pl.ds(off, _CH)]], buf, sem)

        @pl.loop(0, _CH)
        def _(i):
            @pl.loop(0, F, step=16)
            def _(l):
                rows0[i, pl.ds(l, 16)] = jnp.zeros((16,), jnp.float32)

        @pl.loop(0, _RPT // _CH)
        def _(r):
            pltpu.sync_copy(rows0, acc.at[pl.ds(s * _RPT + r * _CH, _CH)])

        rem = _RPT - (_RPT // _CH) * _CH
        pltpu.sync_copy(rows0.at[pl.ds(0, rem)],
                        acc.at[pl.ds(s * _RPT + _RPT - rem, rem)])

        pltpu.sync_copy(src_hbm.at[wid], src_v)
        pltpu.sync_copy(dst_hbm.at[wid], dst_v)
        plsc.subcore_barrier()

        gat(0, rows0, sem0).start()
        gat(1, rows1, sem1).start()

        @pl.loop(0, (_CPT0 - 2) // 2)
        def _(it):
            @pl.when(it < npair)
            def _():
                j = it * 2
                gat(j, rows0, sem0).wait()
                pltpu.sync_copy(rows0, acc.at[dst_v.at[j]], add=True)
                gat(j + 2, rows0, sem0).start()
                gat(j + 1, rows1, sem1).wait()
                pltpu.sync_copy(rows1, acc.at[dst_v.at[j + 1]], add=True)
                gat(j + 3, rows1, sem1).start()

        gat(cpt - 2, rows0, sem0).wait()
        pltpu.sync_copy(rows0, acc.at[dst_v.at[cpt - 2]], add=True)
        gat(cpt - 1, rows1, sem1).wait()
        pltpu.sync_copy(rows1, acc.at[dst_v.at[cpt - 1]], add=True)

        plsc.subcore_barrier()
        pltpu.sync_copy(acc.at[pl.ds(s * _RPT, _RPT)],
                        out_hbm.at[c, pl.ds(s * _RPT, _RPT)])

    return pl.kernel(
        body,
        mesh=mesh,
        out_type=jax.ShapeDtypeStruct((_NC, _NPAD, F), jnp.float32),
        scratch_types=[
            pltpu.VMEM((_SRCW,), jnp.int32),
            pltpu.VMEM((_CPT0, _CH), jnp.int32),
            pltpu.VMEM((_CH, F), jnp.float32),
            pltpu.VMEM((_CH, F), jnp.float32),
            pltpu.VMEM_SHARED((_NPAD, F), jnp.float32),
            pltpu.SemaphoreType.DMA,
            pltpu.SemaphoreType.DMA,
            pltpu.SemaphoreType.DMA,
            pltpu.SemaphoreType.DMA,
        ],
    )


def _count_call():
    mesh = plsc.VectorSubcoreMesh(core_axis_name="c", subcore_axis_name="s")

    def body(dst_hbm, out_hbm, dst_v, cnt_v):
        c = lax.axis_index("c")
        s = lax.axis_index("s")
        wid = c * _NS + s

        @pl.loop(0, _NPAD, step=16)
        def _(i):
            cnt_v[pl.ds(i, 16)] = jnp.zeros((16,), jnp.float32)

        pltpu.sync_copy(dst_hbm.at[wid], dst_v)

        @pl.loop(0, _CPT0)
        def _(j):
            @pl.loop(0, _CH, step=16)
            def _(l):
                idx = dst_v[j, pl.ds(l, 16)]
                plsc.addupdate_scatter(cnt_v, [idx], jnp.ones((16,), jnp.float32))

        pltpu.sync_copy(cnt_v, out_hbm.at[wid])

    return pl.kernel(
        body,
        mesh=mesh,
        out_type=jax.ShapeDtypeStruct((_NW, _NPAD), jnp.float32),
        scratch_types=[
            pltpu.VMEM((_CPT0, _CH), jnp.int32),
            pltpu.VMEM((_NPAD,), jnp.float32),
        ],
        compiler_params=pltpu.CompilerParams(needs_layout_passes=False),
    )


_DN = (((1,), (0,)), ((), ()))


def _prep_body(x_ref, w_ref, b_ref, y_ref, z_ref, *, Fo):
    r = lax.dot_general(x_ref[...], w_ref[...], _DN,
                        preferred_element_type=jnp.float32)
    y_ref[...] = r[:, :Fo]
    z_ref[...] = r[:, Fo:] + b_ref[...]


def _prep_call(x, wcat, b):
    Fi, F2 = wcat.shape
    Fo = F2 // 2
    return pl.pallas_call(
        functools.partial(_prep_body, Fo=Fo),
        grid=(-(-_N // _BR),),
        in_specs=[
            pl.BlockSpec((_BR, Fi), lambda i: (i, 0)),
            pl.BlockSpec((Fi, F2), lambda i: (0, 0)),
            pl.BlockSpec((1, Fo), lambda i: (0, 0)),
        ],
        out_specs=[
            pl.BlockSpec((_BR, Fo), lambda i: (i, 0)),
            pl.BlockSpec((_BR, Fo), lambda i: (i, 0)),
        ],
        out_shape=[jax.ShapeDtypeStruct((_N, Fo), jnp.float32)] * 2,
    )(x, wcat, b)


def _inv_cnt(c_ref):
    cnt = lax.dot_general(c_ref[...], jnp.ones((_NW, 1), jnp.float32),
                          (((0,), (0,)), ((), ())),
                          preferred_element_type=jnp.float32)
    return 1.0 / jnp.maximum(cnt, 1.0)


def _combine(p_ref, c_ref, z_ref):
    return (p_ref[0] + p_ref[1]) * _inv_cnt(c_ref) + z_ref[...]


def _mid_body(p_ref, c_ref, z_ref, w_ref, b_ref, y_ref, z2_ref, *, Fo):
    h = jnp.maximum(_combine(p_ref, c_ref, z_ref), 0.0)
    r = lax.dot_general(h, w_ref[...], _DN, preferred_element_type=jnp.float32)
    y_ref[...] = r[:, :Fo]
    z2_ref[...] = r[:, Fo:] + b_ref[...]


def _mid_call(p, pcnt, z, wcat, b):
    Fi, F2 = wcat.shape
    Fo = F2 // 2
    return pl.pallas_call(
        functools.partial(_mid_body, Fo=Fo),
        grid=(-(-_N // _BR),),
        in_specs=[
            pl.BlockSpec((_NC, _BR, Fi), lambda i: (0, i, 0)),
            pl.BlockSpec((_NW, _BR), lambda i: (0, i)),
            pl.BlockSpec((_BR, Fi), lambda i: (i, 0)),
            pl.BlockSpec((Fi, F2), lambda i: (0, 0)),
            pl.BlockSpec((1, Fo), lambda i: (0, 0)),
        ],
        out_specs=[
            pl.BlockSpec((_BR, Fo), lambda i: (i, 0)),
            pl.BlockSpec((_BR, Fo), lambda i: (i, 0)),
        ],
        out_shape=[jax.ShapeDtypeStruct((_N, Fo), jnp.float32)] * 2,
    )(p, pcnt, z, wcat, b)


def _mid2_body(p_ref, c_ref, z_ref, w_ref, b_ref, h_ref, z3_ref):
    h = jnp.maximum(_combine(p_ref, c_ref, z_ref), 0.0)
    h_ref[...] = h
    z3_ref[...] = lax.dot_general(h, w_ref[...], _DN,
                                  preferred_element_type=jnp.float32) + b_ref[...]


def _mid2_call(p, pcnt, z, wr, b):
    Fi, Fo = wr.shape
    return pl.pallas_call(
        _mid2_body,
        grid=(-(-_N // _BR),),
        in_specs=[
            pl.BlockSpec((_NC, _BR, Fi), lambda i: (0, i, 0)),
            pl.BlockSpec((_NW, _BR), lambda i: (0, i)),
            pl.BlockSpec((_BR, Fi), lambda i: (i, 0)),
            pl.BlockSpec((Fi, Fo), lambda i: (0, 0)),
            pl.BlockSpec((1, Fo), lambda i: (0, 0)),
        ],
        out_specs=[
            pl.BlockSpec((_BR, Fi), lambda i: (i, 0)),
            pl.BlockSpec((_BR, Fo), lambda i: (i, 0)),
        ],
        out_shape=[jax.ShapeDtypeStruct((_N, Fi), jnp.float32),
                   jax.ShapeDtypeStruct((_N, Fo), jnp.float32)],
    )(p, pcnt, z, wr, b)


def _final_body(p_ref, c_ref, z3_ref, w_ref, o_ref):
    mean = (p_ref[0] + p_ref[1]) * _inv_cnt(c_ref)
    o_ref[...] = lax.dot_general(mean, w_ref[...], _DN,
                                 preferred_element_type=jnp.float32) + z3_ref[...]


def _final_call(p, pcnt, z3, wl):
    Fi, Fo = wl.shape
    return pl.pallas_call(
        _final_body,
        grid=(-(-_N // _BR),),
        in_specs=[
            pl.BlockSpec((_NC, _BR, Fi), lambda i: (0, i, 0)),
            pl.BlockSpec((_NW, _BR), lambda i: (0, i)),
            pl.BlockSpec((_BR, Fo), lambda i: (i, 0)),
            pl.BlockSpec((Fi, Fo), lambda i: (0, 0)),
        ],
        out_specs=pl.BlockSpec((_BR, Fo), lambda i: (i, 0)),
        out_shape=jax.ShapeDtypeStruct((_N, Fo), jnp.float32),
    )(p, pcnt, z3, wl)


def kernel(x, edge_index, Wl1, bl1, Wr1, Wl2, bl2, Wr2, Wl3, bl3, Wr3):
    src = edge_index[0].astype(jnp.int32)
    dst = edge_index[1].astype(jnp.int32)
    s0 = src[:_E0].reshape(_NS, _SRCW)
    s1 = jnp.concatenate(
        [src[_E0:].reshape(_NS, _E1PT),
         jnp.zeros((_NS, _SRCW - _E1PT), jnp.int32)], axis=1)
    srcp = jnp.concatenate([s0, s1], axis=0)
    d0 = dst[:_E0].reshape(_NS, _CPT0, _CH)
    d1 = jnp.concatenate(
        [dst[_E0:].reshape(_NS, _E1PT),
         jnp.full((_NS, _SRCW - _E1PT), _N, jnp.int32)],
        axis=1).reshape(_NS, _CPT0, _CH)
    dstp = jnp.concatenate([d0, d1], axis=0)

    w1 = jnp.concatenate([Wl1, Wr1], axis=1)
    w2 = jnp.concatenate([Wl2, Wr2], axis=1)
    b1, b2, b3 = (b.reshape(1, -1) for b in (bl1, bl2, bl3))

    pcnt = _count_call()(dstp)
    y1, z1 = _prep_call(x, w1, b1)
    p1 = _seg_sum_call(128)(y1, srcp, dstp)
    y2, z2 = _mid_call(p1, pcnt, z1, w2, b2)
    p2 = _seg_sum_call(128)(y2, srcp, dstp)
    h2, z3 = _mid2_call(p2, pcnt, z2, Wr3, b3)
    p3 = _seg_sum_call(128)(h2, srcp, dstp)
    return _final_call(p3, pcnt, z3, Wl3)
